# baseline (speedup 1.0000x reference)
import functools

import jax
import jax.numpy as jnp
from jax import lax
from jax.experimental import pallas as pl
from jax.experimental.pallas import tpu as pltpu
from jax.experimental.pallas import tpu_sc as plsc

_EMB = 32
_B = 16384
_NF = 19
_NC = 2
_NS = 16
_NW = _NC * _NS
_BPW = _B // _NW
_CH = 128
_NCH = _BPW // _CH
_NBUF = 12


def _body(xt_hbm, *refs):
    tables = refs[:_NF]
    out_hbm = refs[_NF]
    rest = refs[_NF + 1:]
    idx_v = rest[:_NF]
    rows_v = rest[_NF:_NF + _NBUF]
    isem, gsem, wsem = rest[_NF + _NBUF:]
    wid = lax.axis_index("s") * _NC + lax.axis_index("c")
    base = wid * _BPW

    idescr = [
        pltpu.async_copy(xt_hbm.at[pl.ds(i * _B + base, _BPW)], idx_v[i], isem)
        for i in range(_NF)
    ]
    for d in idescr:
        d.wait()

    def gather(t):
        i, c = t // _NCH, t % _NCH
        return pltpu.async_copy(
            tables[i].at[idx_v[i].at[pl.ds(c * _CH, _CH)]],
            rows_v[t % _NBUF], gsem)

    def write(t):
        i, c = t // _NCH, t % _NCH
        return pltpu.async_copy(
            rows_v[t % _NBUF],
            out_hbm.at[pl.ds(base + c * _CH, _CH), pl.ds(i * _EMB, _EMB)],
            wsem)

    ntask = _NF * _NCH
    gd = [gather(t) for t in range(_NBUF)]
    wd = []
    for t in range(ntask):
        gd[t].wait()
        wd.append(write(t))
        j = t + _NBUF
        if j < ntask:
            wd[t].wait()
            gd.append(gather(j))
    for t in range(ntask - _NBUF, ntask):
        wd[t].wait()


_sc_lookup = functools.partial(
    pl.kernel,
    out_type=jax.ShapeDtypeStruct((_B, _NF * _EMB), jnp.float32),
    mesh=plsc.VectorSubcoreMesh(core_axis_name="c", subcore_axis_name="s"),
    compiler_params=pltpu.CompilerParams(use_tc_tiling_on_sc=False),
    scratch_types=(
        [pltpu.VMEM((_BPW,), jnp.int32) for _ in range(_NF)]
        + [pltpu.VMEM((_CH, _EMB), jnp.float32) for _ in range(_NBUF)]
        + [pltpu.SemaphoreType.DMA] * 3
    ),
)(_body)


def kernel(x, W0, W1, W2, W3, W4, W5, W6, W7, W8, W9, W10, W11, W12, W13,
           W14, W15, W16, W17, W18):
    xt = x.T.reshape(-1)
    out = _sc_lookup(xt, W0, W1, W2, W3, W4, W5, W6, W7, W8, W9, W10, W11,
                     W12, W13, W14, W15, W16, W17, W18)
    return out.reshape(_B, _NF, _EMB)
